# Initial kernel scaffold; baseline (speedup 1.0000x reference)
import jax
import jax.numpy as jnp
from jax.experimental import pallas as pl


def kernel(node_embs, edge_embs, edge_index, e_vid, hids, rids, tids, is_head_pred, WO1, bO1, WI1, bI1, WS1, bS1, bn1w, bn1b, Wr1, br1, WO2, bO2, WI2, bI2, WS2, bS2, bn2w, bn2b):
    raise NotImplementedError("write your pallas kernel here")


# speedup vs baseline: 3.4415x; 3.4415x over previous
import functools

import numpy as np

import jax
import jax.numpy as jnp
from jax import lax
from jax.experimental import pallas as pl
from jax.experimental.pallas import tpu as pltpu
from jax.experimental.pallas import tpu_sc as plsc

N_NODES = 10000
N_EDGES = 320000
D = 128
N_REL = 500
B = 16
GAMMA = 9.0

CH = 80
ROWS = N_EDGES // CH
NSUB = 16
ROWS_PER_TILE = ROWS // NSUB
NACC = 10240
NSLICE = NACC // NSUB
CNTW = 16

_BN_SCALE = 1.0 / float(np.sqrt(np.float32(1.0 + 1e-5), dtype=np.float32))


def _make_sc_scatter():
    mesh = plsc.VectorSubcoreMesh(core_axis_name="c", subcore_axis_name="s")
    out_type = [jax.ShapeDtypeStruct((2, NACC, D), jnp.float32)]
    scratch = [
        pltpu.VMEM_SHARED((NACC, D), jnp.float32),
        pltpu.VMEM((CH,), jnp.int32),
        pltpu.VMEM((CH,), jnp.int32),
        pltpu.VMEM((CH,), jnp.int32),
        pltpu.VMEM((CH, D), jnp.float32),
        pltpu.VMEM((CH, D), jnp.float32),
        pltpu.SemaphoreType.DMA,
        pltpu.SemaphoreType.DMA,
    ]

    @functools.partial(pl.kernel, mesh=mesh, out_type=out_type,
                       scratch_types=scratch)
    def sc_fn(x_hbm, tn_hbm, gidx_hbm, sidx_hbm, vid_hbm, zrows_hbm, s_out,
              acc, g_idx, s_idx, v_idx, xbuf, tbuf, sem1, sem2):
        cid = lax.axis_index("c")
        sid = lax.axis_index("s")

        pltpu.sync_copy(zrows_hbm, acc.at[pl.ds(sid * NSLICE, NSLICE)])
        plsc.subcore_barrier()

        def body(it, carry):
            ebase = (sid * ROWS_PER_TILE + it) * CH
            gbase = cid * N_EDGES + ebase
            pltpu.sync_copy(gidx_hbm.at[pl.ds(gbase, CH)], g_idx)
            pltpu.sync_copy(sidx_hbm.at[pl.ds(gbase, CH)], s_idx)
            pltpu.sync_copy(vid_hbm.at[pl.ds(ebase, CH)], v_idx)
            cp1 = pltpu.async_copy(x_hbm.at[g_idx], xbuf, sem1)
            cp2 = pltpu.async_copy(tn_hbm.at[v_idx], tbuf, sem2)
            cp1.wait()
            cp2.wait()
            pltpu.async_copy(xbuf, acc.at[s_idx], sem1, add=True).wait()
            pltpu.async_copy(tbuf, acc.at[s_idx], sem2, add=True).wait()
            return carry
        lax.fori_loop(0, ROWS_PER_TILE, body, 0)

        plsc.subcore_barrier()
        sl = pl.ds(sid * NSLICE, NSLICE)
        pltpu.sync_copy(acc.at[sl], s_out.at[cid, sl])

    return sc_fn


def _make_sc_counts():
    mesh = plsc.VectorSubcoreMesh(core_axis_name="c", subcore_axis_name="s")
    out_type = [jax.ShapeDtypeStruct((2, NACC, D), jnp.float32)]
    scratch = [
        pltpu.VMEM_SHARED((NACC, D), jnp.float32),
        pltpu.VMEM((CH,), jnp.int32),
        pltpu.VMEM((CH, D), jnp.float32),
        pltpu.SemaphoreType.DMA,
    ]

    @functools.partial(pl.kernel, mesh=mesh, out_type=out_type,
                       scratch_types=scratch)
    def sc_fn(sidx_hbm, ones_hbm, zrows_hbm, c_out, cnt, s_idx, ones, sem):
        cid = lax.axis_index("c")
        sid = lax.axis_index("s")

        pltpu.sync_copy(zrows_hbm, cnt.at[pl.ds(sid * NSLICE, NSLICE)])
        pltpu.sync_copy(ones_hbm, ones)
        plsc.subcore_barrier()

        def body(it, carry):
            base = cid * N_EDGES + (sid * ROWS_PER_TILE + it) * CH
            pltpu.sync_copy(sidx_hbm.at[pl.ds(base, CH)], s_idx)
            pltpu.async_copy(ones, cnt.at[s_idx], sem, add=True).wait()
            return carry
        lax.fori_loop(0, ROWS_PER_TILE, body, 0)

        plsc.subcore_barrier()
        sl = pl.ds(sid * NSLICE, NSLICE)
        pltpu.sync_copy(cnt.at[sl], c_out.at[cid, sl])

    return sc_fn


_sc_layer = _make_sc_scatter()
_sc_counts = _make_sc_counts()


def _tables_body(ee_ref, wr_ref, br_ref, t1_ref, t2_ref):
    ee = ee_ref[...]
    t1_ref[...] = -ee
    t2_ref[...] = -(jnp.dot(ee, wr_ref[...], preferred_element_type=jnp.float32)
                    + br_ref[...])


def _rel_tables(edge_embs, Wr1, br1):
    return pl.pallas_call(
        _tables_body,
        out_shape=(jax.ShapeDtypeStruct((N_REL, D), jnp.float32),
                   jax.ShapeDtypeStruct((N_REL, D), jnp.float32)),
    )(edge_embs, Wr1, br1.reshape(1, D))


LBLK = 512


def _layer_body(apply_tanh, sd_ref, ss_ref, cd_ref, cs_ref, x_ref,
                wo_ref, bo_ref, wi_ref, bi_ref, ws_ref, bs_ref,
                bnw_ref, bnb_ref, o_ref):
    deg_d = jnp.maximum(cd_ref[:, 0:1], 1.0)
    deg_s = jnp.maximum(cs_ref[:, 0:1], 1.0)
    ho = sd_ref[...] / deg_d
    hi = ss_ref[...] / deg_s
    h = (jnp.dot(ho, wo_ref[...], preferred_element_type=jnp.float32) + bo_ref[...]
         + jnp.dot(hi, wi_ref[...], preferred_element_type=jnp.float32) + bi_ref[...]
         + jnp.dot(x_ref[...], ws_ref[...], preferred_element_type=jnp.float32)
         + bs_ref[...]) * (1.0 / 3.0)
    h = h * (bnw_ref[...] * _BN_SCALE) + bnb_ref[...]
    o_ref[...] = jnp.tanh(h) if apply_tanh else h


def _layer_tc(Sd, Ss, Cd, Cs, X, WO, bO, WI, bI, WS, bS, bnw, bnb, apply_tanh):
    grid = (NACC // LBLK,)
    row_spec = pl.BlockSpec((LBLK, D), lambda i: (i, 0))
    cnt_spec = pl.BlockSpec((LBLK, CNTW), lambda i: (i, 0))
    w_spec = pl.BlockSpec((D, D), lambda i: (0, 0))
    b_spec = pl.BlockSpec((1, D), lambda i: (0, 0))
    return pl.pallas_call(
        functools.partial(_layer_body, apply_tanh),
        grid=grid,
        in_specs=[row_spec, row_spec, cnt_spec, cnt_spec, row_spec,
                  w_spec, b_spec, w_spec, b_spec, w_spec, b_spec,
                  b_spec, b_spec],
        out_specs=row_spec,
        out_shape=jax.ShapeDtypeStruct((NACC, D), jnp.float32),
    )(Sd, Ss, Cd, Cs, X,
      WO, bO.reshape(1, D), WI, bI.reshape(1, D), WS, bS.reshape(1, D),
      bnw.reshape(1, D), bnb.reshape(1, D))


NPAD = 640
SBLK = 128
SGRID = NPAD // SBLK


def _score_body(hn_blk_ref, hn_full_ref, ee_ref, wr_ref, br_ref,
                hid_ref, rid_ref, tid_ref, ihp_ref, o_ref, c_ref):
    q = pl.program_id(0)
    j = pl.program_id(1)

    @pl.when(jnp.logical_and(q == 0, j == 0))
    def _():
        heads = jnp.concatenate(
            [hn_full_ref[pl.ds(hid_ref[0, b], 1), :] for b in range(B)], axis=0)
        tails = jnp.concatenate(
            [hn_full_ref[pl.ds(tid_ref[0, b], 1), :] for b in range(B)], axis=0)
        rrows = jnp.concatenate(
            [ee_ref[pl.ds(rid_ref[0, b], 1), :] for b in range(B)], axis=0)
        rels = jnp.dot(rrows, wr_ref[...],
                       preferred_element_type=jnp.float32) + br_ref[...]
        c_ref[...] = jnp.where(ihp_ref[0, 0] == 1, rels - tails,
                               -(heads + rels))

    cq = c_ref[pl.ds(q, 1), :]
    dist = jnp.sum(jnp.abs(hn_blk_ref[...] + cq), axis=1)
    o_ref[0, 0, :] = jax.nn.sigmoid(GAMMA - dist)


def _score_tc(hn_pad, edge_embs, Wr1, br1, hids, rids, tids, ihp):
    grid = (B, SGRID)
    smem = pl.BlockSpec(memory_space=pltpu.SMEM)
    return pl.pallas_call(
        _score_body,
        grid=grid,
        in_specs=[
            pl.BlockSpec((SBLK, D), lambda q, j: (q * SGRID + j, 0)),
            pl.BlockSpec((B * NPAD, D), lambda q, j: (0, 0)),
            pl.BlockSpec((N_REL, D), lambda q, j: (0, 0)),
            pl.BlockSpec((D, D), lambda q, j: (0, 0)),
            pl.BlockSpec((1, D), lambda q, j: (0, 0)),
            smem, smem, smem, smem,
        ],
        out_specs=pl.BlockSpec((1, 1, SBLK), lambda q, j: (q, 0, j)),
        out_shape=jax.ShapeDtypeStruct((B, 1, NPAD), jnp.float32),
        scratch_shapes=[pltpu.VMEM((B, D), jnp.float32)],
    )(hn_pad, hn_pad, edge_embs, Wr1, br1.reshape(1, D),
      hids.reshape(1, B), rids.reshape(1, B), tids.reshape(1, B),
      jnp.asarray(ihp, jnp.int32).reshape(1, 1))


def kernel(node_embs, edge_embs, edge_index, e_vid, hids, rids, tids,
           is_head_pred, WO1, bO1, WI1, bI1, WS1, bS1, bn1w, bn1b, Wr1, br1,
           WO2, bO2, WI2, bI2, WS2, bS2, bn2w, bn2b):
    src1d = edge_index[0].reshape(N_EDGES)
    dst1d = edge_index[1].reshape(N_EDGES)
    vid1d = e_vid.reshape(N_EDGES)
    gidx2 = jnp.concatenate([src1d, dst1d])
    sidx2 = jnp.concatenate([dst1d, src1d])

    negT1, negT2 = _rel_tables(edge_embs, Wr1, br1)

    x0 = jnp.pad(node_embs, ((0, NACC - N_NODES), (0, 0)))
    zrows = jnp.zeros((NSLICE, D), jnp.float32)
    ones_rows = jnp.ones((CH, D), jnp.float32)

    Cb = _sc_counts(sidx2, ones_rows, zrows)[0]
    Cd = Cb[0, :, :CNTW]
    Cs = Cb[1, :, :CNTW]

    S1 = _sc_layer(x0, negT1, gidx2, sidx2, vid1d, zrows)[0]
    h1 = _layer_tc(S1[0], S1[1], Cd, Cs, x0,
                   WO1, bO1, WI1, bI1, WS1, bS1, bn1w, bn1b, apply_tanh=True)

    S2 = _sc_layer(h1, negT2, gidx2, sidx2, vid1d, zrows)[0]
    hn = _layer_tc(S2[0], S2[1], Cd, Cs, h1,
                   WO2, bO2, WI2, bI2, WS2, bS2, bn2w, bn2b,
                   apply_tanh=False)[:N_NODES]

    hn_pad = jnp.pad(hn.reshape(B, N_NODES // B, D),
                     ((0, 0), (0, NPAD - N_NODES // B), (0, 0))
                     ).reshape(B * NPAD, D)
    P = _score_tc(hn_pad, edge_embs, Wr1, br1, hids, rids, tids,
                  is_head_pred).reshape(B, NPAD)
    score = jnp.repeat(P[:, :N_NODES // B], B, axis=1)
    return hn, score


# speedup vs baseline: 5.0987x; 1.4815x over previous
import functools

import numpy as np

import jax
import jax.numpy as jnp
from jax import lax
from jax.experimental import pallas as pl
from jax.experimental.pallas import tpu as pltpu
from jax.experimental.pallas import tpu_sc as plsc

N_NODES = 10000
N_EDGES = 320000
D = 128
N_REL = 500
B = 16
GAMMA = 9.0

CH = 80
ROWS = N_EDGES // CH
NSUB = 16
ROWS_PER_TILE = ROWS // NSUB
NACC = 10240
NSLICE = NACC // NSUB
CNTW = 16

_BN_SCALE = 1.0 / float(np.sqrt(np.float32(1.0 + 1e-5), dtype=np.float32))


def _make_sc_scatter():
    mesh = plsc.VectorSubcoreMesh(core_axis_name="c", subcore_axis_name="s")
    out_type = [jax.ShapeDtypeStruct((2, NACC, D), jnp.float32)]
    scratch = [
        pltpu.VMEM_SHARED((NACC, D), jnp.float32),
        pltpu.VMEM((CH,), jnp.int32),
        pltpu.VMEM((CH,), jnp.int32),
        pltpu.VMEM((CH,), jnp.int32),
        pltpu.VMEM((CH,), jnp.int32),
        pltpu.VMEM((CH,), jnp.int32),
        pltpu.VMEM((CH,), jnp.int32),
        pltpu.VMEM((CH, D), jnp.float32),
        pltpu.VMEM((CH, D), jnp.float32),
        pltpu.VMEM((CH, D), jnp.float32),
        pltpu.VMEM((CH, D), jnp.float32),
        pltpu.SemaphoreType.DMA,
        pltpu.SemaphoreType.DMA,
    ]

    @functools.partial(pl.kernel, mesh=mesh, out_type=out_type,
                       scratch_types=scratch)
    def sc_fn(x_hbm, tn_hbm, gidx_hbm, sidx_hbm, vid_hbm, zrows_hbm, s_out,
              acc, g_idx0, s_idx0, v_idx0, g_idx1, s_idx1, v_idx1,
              xbuf0, tbuf0, xbuf1, tbuf1, sem1, sem2):
        cid = lax.axis_index("c")
        sid = lax.axis_index("s")

        pltpu.sync_copy(zrows_hbm, acc.at[pl.ds(sid * NSLICE, NSLICE)])
        plsc.subcore_barrier()

        def body(it, carry):
            eb0 = (sid * ROWS_PER_TILE + 2 * it) * CH
            eb1 = eb0 + CH
            gb0 = cid * N_EDGES + eb0
            gb1 = gb0 + CH
            ws = [
                pltpu.async_copy(gidx_hbm.at[pl.ds(gb0, CH)], g_idx0, sem1),
                pltpu.async_copy(sidx_hbm.at[pl.ds(gb0, CH)], s_idx0, sem1),
                pltpu.async_copy(vid_hbm.at[pl.ds(eb0, CH)], v_idx0, sem1),
                pltpu.async_copy(gidx_hbm.at[pl.ds(gb1, CH)], g_idx1, sem2),
                pltpu.async_copy(sidx_hbm.at[pl.ds(gb1, CH)], s_idx1, sem2),
                pltpu.async_copy(vid_hbm.at[pl.ds(eb1, CH)], v_idx1, sem2),
            ]
            for w in ws:
                w.wait()
            gs = [
                pltpu.async_copy(x_hbm.at[g_idx0], xbuf0, sem1),
                pltpu.async_copy(tn_hbm.at[v_idx0], tbuf0, sem1),
                pltpu.async_copy(x_hbm.at[g_idx1], xbuf1, sem2),
                pltpu.async_copy(tn_hbm.at[v_idx1], tbuf1, sem2),
            ]
            for g in gs:
                g.wait()
            ss = [
                pltpu.async_copy(xbuf0, acc.at[s_idx0], sem1, add=True),
                pltpu.async_copy(tbuf0, acc.at[s_idx0], sem1, add=True),
                pltpu.async_copy(xbuf1, acc.at[s_idx1], sem2, add=True),
                pltpu.async_copy(tbuf1, acc.at[s_idx1], sem2, add=True),
            ]
            for s in ss:
                s.wait()
            return carry
        lax.fori_loop(0, ROWS_PER_TILE // 2, body, 0)

        plsc.subcore_barrier()
        sl = pl.ds(sid * NSLICE, NSLICE)
        pltpu.sync_copy(acc.at[sl], s_out.at[cid, sl])

    return sc_fn


def _make_sc_counts():
    mesh = plsc.VectorSubcoreMesh(core_axis_name="c", subcore_axis_name="s")
    out_type = [jax.ShapeDtypeStruct((2, NACC, D), jnp.float32)]
    scratch = [
        pltpu.VMEM_SHARED((NACC, D), jnp.float32),
        pltpu.VMEM((CH,), jnp.int32),
        pltpu.VMEM((CH,), jnp.int32),
        pltpu.VMEM((CH, D), jnp.float32),
        pltpu.SemaphoreType.DMA,
        pltpu.SemaphoreType.DMA,
    ]

    @functools.partial(pl.kernel, mesh=mesh, out_type=out_type,
                       scratch_types=scratch)
    def sc_fn(sidx_hbm, ones_hbm, zrows_hbm, c_out, cnt, s_idx0, s_idx1,
              ones, sem1, sem2):
        cid = lax.axis_index("c")
        sid = lax.axis_index("s")

        pltpu.sync_copy(zrows_hbm, cnt.at[pl.ds(sid * NSLICE, NSLICE)])
        pltpu.sync_copy(ones_hbm, ones)
        plsc.subcore_barrier()

        def body(it, carry):
            b0 = cid * N_EDGES + (sid * ROWS_PER_TILE + 2 * it) * CH
            b1 = b0 + CH
            i0 = pltpu.async_copy(sidx_hbm.at[pl.ds(b0, CH)], s_idx0, sem1)
            i1 = pltpu.async_copy(sidx_hbm.at[pl.ds(b1, CH)], s_idx1, sem2)
            i0.wait()
            i1.wait()
            w0 = pltpu.async_copy(ones, cnt.at[s_idx0], sem1, add=True)
            w1 = pltpu.async_copy(ones, cnt.at[s_idx1], sem2, add=True)
            w0.wait()
            w1.wait()
            return carry
        lax.fori_loop(0, ROWS_PER_TILE // 2, body, 0)

        plsc.subcore_barrier()
        sl = pl.ds(sid * NSLICE, NSLICE)
        pltpu.sync_copy(cnt.at[sl], c_out.at[cid, sl])

    return sc_fn


_sc_layer = _make_sc_scatter()
_sc_counts = _make_sc_counts()


def _tables_body(ee_ref, wr_ref, br_ref, t1_ref, t2_ref):
    ee = ee_ref[...]
    t1_ref[...] = -ee
    t2_ref[...] = -(jnp.dot(ee, wr_ref[...], preferred_element_type=jnp.float32)
                    + br_ref[...])


def _rel_tables(edge_embs, Wr1, br1):
    return pl.pallas_call(
        _tables_body,
        out_shape=(jax.ShapeDtypeStruct((N_REL, D), jnp.float32),
                   jax.ShapeDtypeStruct((N_REL, D), jnp.float32)),
    )(edge_embs, Wr1, br1.reshape(1, D))


LBLK = 512


def _layer_body(apply_tanh, sd_ref, ss_ref, cd_ref, cs_ref, x_ref,
                wo_ref, bo_ref, wi_ref, bi_ref, ws_ref, bs_ref,
                bnw_ref, bnb_ref, o_ref):
    deg_d = jnp.maximum(cd_ref[:, 0:1], 1.0)
    deg_s = jnp.maximum(cs_ref[:, 0:1], 1.0)
    ho = sd_ref[...] / deg_d
    hi = ss_ref[...] / deg_s
    h = (jnp.dot(ho, wo_ref[...], preferred_element_type=jnp.float32) + bo_ref[...]
         + jnp.dot(hi, wi_ref[...], preferred_element_type=jnp.float32) + bi_ref[...]
         + jnp.dot(x_ref[...], ws_ref[...], preferred_element_type=jnp.float32)
         + bs_ref[...]) * (1.0 / 3.0)
    h = h * (bnw_ref[...] * _BN_SCALE) + bnb_ref[...]
    o_ref[...] = jnp.tanh(h) if apply_tanh else h


def _layer_tc(Sd, Ss, Cd, Cs, X, WO, bO, WI, bI, WS, bS, bnw, bnb, apply_tanh):
    grid = (NACC // LBLK,)
    row_spec = pl.BlockSpec((LBLK, D), lambda i: (i, 0))
    cnt_spec = pl.BlockSpec((LBLK, CNTW), lambda i: (i, 0))
    w_spec = pl.BlockSpec((D, D), lambda i: (0, 0))
    b_spec = pl.BlockSpec((1, D), lambda i: (0, 0))
    return pl.pallas_call(
        functools.partial(_layer_body, apply_tanh),
        grid=grid,
        in_specs=[row_spec, row_spec, cnt_spec, cnt_spec, row_spec,
                  w_spec, b_spec, w_spec, b_spec, w_spec, b_spec,
                  b_spec, b_spec],
        out_specs=row_spec,
        out_shape=jax.ShapeDtypeStruct((NACC, D), jnp.float32),
    )(Sd, Ss, Cd, Cs, X,
      WO, bO.reshape(1, D), WI, bI.reshape(1, D), WS, bS.reshape(1, D),
      bnw.reshape(1, D), bnb.reshape(1, D))


NPAD = 640
SBLK = 128
SGRID = NPAD // SBLK


def _score_body(hn_blk_ref, hn_full_ref, ee_ref, wr_ref, br_ref,
                hid_ref, rid_ref, tid_ref, ihp_ref, o_ref, c_ref):
    q = pl.program_id(0)
    j = pl.program_id(1)

    @pl.when(jnp.logical_and(q == 0, j == 0))
    def _():
        heads = jnp.concatenate(
            [hn_full_ref[pl.ds(hid_ref[0, b], 1), :] for b in range(B)], axis=0)
        tails = jnp.concatenate(
            [hn_full_ref[pl.ds(tid_ref[0, b], 1), :] for b in range(B)], axis=0)
        rrows = jnp.concatenate(
            [ee_ref[pl.ds(rid_ref[0, b], 1), :] for b in range(B)], axis=0)
        rels = jnp.dot(rrows, wr_ref[...],
                       preferred_element_type=jnp.float32) + br_ref[...]
        c_ref[...] = jnp.where(ihp_ref[0, 0] == 1, rels - tails,
                               -(heads + rels))

    cq = c_ref[pl.ds(q, 1), :]
    dist = jnp.sum(jnp.abs(hn_blk_ref[...] + cq), axis=1)
    o_ref[0, 0, :] = jax.nn.sigmoid(GAMMA - dist)


def _score_tc(hn_pad, edge_embs, Wr1, br1, hids, rids, tids, ihp):
    grid = (B, SGRID)
    smem = pl.BlockSpec(memory_space=pltpu.SMEM)
    return pl.pallas_call(
        _score_body,
        grid=grid,
        in_specs=[
            pl.BlockSpec((SBLK, D), lambda q, j: (q * SGRID + j, 0)),
            pl.BlockSpec((B * NPAD, D), lambda q, j: (0, 0)),
            pl.BlockSpec((N_REL, D), lambda q, j: (0, 0)),
            pl.BlockSpec((D, D), lambda q, j: (0, 0)),
            pl.BlockSpec((1, D), lambda q, j: (0, 0)),
            smem, smem, smem, smem,
        ],
        out_specs=pl.BlockSpec((1, 1, SBLK), lambda q, j: (q, 0, j)),
        out_shape=jax.ShapeDtypeStruct((B, 1, NPAD), jnp.float32),
        scratch_shapes=[pltpu.VMEM((B, D), jnp.float32)],
    )(hn_pad, hn_pad, edge_embs, Wr1, br1.reshape(1, D),
      hids.reshape(1, B), rids.reshape(1, B), tids.reshape(1, B),
      jnp.asarray(ihp, jnp.int32).reshape(1, 1))


def kernel(node_embs, edge_embs, edge_index, e_vid, hids, rids, tids,
           is_head_pred, WO1, bO1, WI1, bI1, WS1, bS1, bn1w, bn1b, Wr1, br1,
           WO2, bO2, WI2, bI2, WS2, bS2, bn2w, bn2b):
    src1d = edge_index[0].reshape(N_EDGES)
    dst1d = edge_index[1].reshape(N_EDGES)
    vid1d = e_vid.reshape(N_EDGES)
    gidx2 = jnp.concatenate([src1d, dst1d])
    sidx2 = jnp.concatenate([dst1d, src1d])

    negT1, negT2 = _rel_tables(edge_embs, Wr1, br1)

    x0 = jnp.pad(node_embs, ((0, NACC - N_NODES), (0, 0)))
    zrows = jnp.zeros((NSLICE, D), jnp.float32)
    ones_rows = jnp.ones((CH, D), jnp.float32)

    Cb = _sc_counts(sidx2, ones_rows, zrows)[0]
    Cd = Cb[0, :, :CNTW]
    Cs = Cb[1, :, :CNTW]

    S1 = _sc_layer(x0, negT1, gidx2, sidx2, vid1d, zrows)[0]
    h1 = _layer_tc(S1[0], S1[1], Cd, Cs, x0,
                   WO1, bO1, WI1, bI1, WS1, bS1, bn1w, bn1b, apply_tanh=True)

    S2 = _sc_layer(h1, negT2, gidx2, sidx2, vid1d, zrows)[0]
    hn = _layer_tc(S2[0], S2[1], Cd, Cs, h1,
                   WO2, bO2, WI2, bI2, WS2, bS2, bn2w, bn2b,
                   apply_tanh=False)[:N_NODES]

    hn_pad = jnp.pad(hn.reshape(B, N_NODES // B, D),
                     ((0, 0), (0, NPAD - N_NODES // B), (0, 0))
                     ).reshape(B * NPAD, D)
    P = _score_tc(hn_pad, edge_embs, Wr1, br1, hids, rids, tids,
                  is_head_pred).reshape(B, NPAD)
    score = jnp.repeat(P[:, :N_NODES // B], B, axis=1)
    return hn, score


# speedup vs baseline: 5.2208x; 1.0240x over previous
import functools

import numpy as np

import jax
import jax.numpy as jnp
from jax import lax
from jax.experimental import pallas as pl
from jax.experimental.pallas import tpu as pltpu
from jax.experimental.pallas import tpu_sc as plsc

N_NODES = 10000
N_EDGES = 320000
D = 128
N_REL = 500
B = 16
GAMMA = 9.0

CH = 80
ROWS = N_EDGES // CH
NSUB = 16
ROWS_PER_TILE = ROWS // NSUB
NACC = 10240
NSLICE = NACC // NSUB
CNTW = 16

_BN_SCALE = 1.0 / float(np.sqrt(np.float32(1.0 + 1e-5), dtype=np.float32))


def _make_sc_scatter(NSLOT):
    mesh = plsc.VectorSubcoreMesh(core_axis_name="c", subcore_axis_name="s")
    out_type = [jax.ShapeDtypeStruct((2, NACC, D), jnp.float32)]
    scratch = (
        [pltpu.VMEM_SHARED((NACC, D), jnp.float32)]
        + [pltpu.VMEM((CH,), jnp.int32)] * (3 * NSLOT)
        + [pltpu.VMEM((CH, D), jnp.float32)] * (2 * NSLOT)
        + [pltpu.SemaphoreType.DMA, pltpu.SemaphoreType.DMA]
    )

    @functools.partial(pl.kernel, mesh=mesh, out_type=out_type,
                       scratch_types=scratch)
    def sc_fn(x_hbm, tn_hbm, gidx_hbm, sidx_hbm, vid_hbm, zrows_hbm, s_out,
              acc, *rest):
        g_idx = rest[0:NSLOT]
        s_idx = rest[NSLOT:2 * NSLOT]
        v_idx = rest[2 * NSLOT:3 * NSLOT]
        xbuf = rest[3 * NSLOT:4 * NSLOT]
        tbuf = rest[4 * NSLOT:5 * NSLOT]
        sems = rest[5 * NSLOT:]
        cid = lax.axis_index("c")
        sid = lax.axis_index("s")

        pltpu.sync_copy(zrows_hbm, acc.at[pl.ds(sid * NSLICE, NSLICE)])
        plsc.subcore_barrier()

        def body(it, carry):
            eb = [(sid * ROWS_PER_TILE + NSLOT * it + k) * CH
                  for k in range(NSLOT)]
            gb = [cid * N_EDGES + e for e in eb]
            ws = []
            for k in range(NSLOT):
                sm = sems[k % 2]
                ws += [
                    pltpu.async_copy(gidx_hbm.at[pl.ds(gb[k], CH)], g_idx[k], sm),
                    pltpu.async_copy(sidx_hbm.at[pl.ds(gb[k], CH)], s_idx[k], sm),
                    pltpu.async_copy(vid_hbm.at[pl.ds(eb[k], CH)], v_idx[k], sm),
                ]
            for w in ws:
                w.wait()
            gs = []
            for k in range(NSLOT):
                sm = sems[k % 2]
                gs += [
                    pltpu.async_copy(x_hbm.at[g_idx[k]], xbuf[k], sm),
                    pltpu.async_copy(tn_hbm.at[v_idx[k]], tbuf[k], sm),
                ]
            for g in gs:
                g.wait()
            ss = []
            for k in range(NSLOT):
                sm = sems[k % 2]
                ss += [
                    pltpu.async_copy(xbuf[k], acc.at[s_idx[k]], sm, add=True),
                    pltpu.async_copy(tbuf[k], acc.at[s_idx[k]], sm, add=True),
                ]
            for s in ss:
                s.wait()
            return carry
        lax.fori_loop(0, ROWS_PER_TILE // NSLOT, body, 0)

        plsc.subcore_barrier()
        sl = pl.ds(sid * NSLICE, NSLICE)
        pltpu.sync_copy(acc.at[sl], s_out.at[cid, sl])

    return sc_fn


def _make_sc_counts(NSLOT):
    mesh = plsc.VectorSubcoreMesh(core_axis_name="c", subcore_axis_name="s")
    out_type = [jax.ShapeDtypeStruct((2, NACC, D), jnp.float32)]
    scratch = (
        [pltpu.VMEM_SHARED((NACC, D), jnp.float32)]
        + [pltpu.VMEM((CH,), jnp.int32)] * NSLOT
        + [pltpu.VMEM((CH, D), jnp.float32)]
        + [pltpu.SemaphoreType.DMA, pltpu.SemaphoreType.DMA]
    )

    @functools.partial(pl.kernel, mesh=mesh, out_type=out_type,
                       scratch_types=scratch)
    def sc_fn(sidx_hbm, ones_hbm, zrows_hbm, c_out, cnt, *rest):
        s_idx = rest[0:NSLOT]
        ones = rest[NSLOT]
        sems = rest[NSLOT + 1:]
        cid = lax.axis_index("c")
        sid = lax.axis_index("s")

        pltpu.sync_copy(zrows_hbm, cnt.at[pl.ds(sid * NSLICE, NSLICE)])
        pltpu.sync_copy(ones_hbm, ones)
        plsc.subcore_barrier()

        def body(it, carry):
            bs = [cid * N_EDGES + (sid * ROWS_PER_TILE + NSLOT * it + k) * CH
                  for k in range(NSLOT)]
            iws = [pltpu.async_copy(sidx_hbm.at[pl.ds(bs[k], CH)], s_idx[k],
                                    sems[k % 2]) for k in range(NSLOT)]
            for w in iws:
                w.wait()
            sws = [pltpu.async_copy(ones, cnt.at[s_idx[k]], sems[k % 2],
                                    add=True) for k in range(NSLOT)]
            for s in sws:
                s.wait()
            return carry
        lax.fori_loop(0, ROWS_PER_TILE // NSLOT, body, 0)

        plsc.subcore_barrier()
        sl = pl.ds(sid * NSLICE, NSLICE)
        pltpu.sync_copy(cnt.at[sl], c_out.at[cid, sl])

    return sc_fn


_sc_layer = _make_sc_scatter(2)
_sc_counts = _make_sc_counts(5)


def _tables_body(ee_ref, wr_ref, br_ref, t1_ref, t2_ref):
    ee = ee_ref[...]
    t1_ref[...] = -ee
    t2_ref[...] = -(jnp.dot(ee, wr_ref[...], preferred_element_type=jnp.float32)
                    + br_ref[...])


def _rel_tables(edge_embs, Wr1, br1):
    return pl.pallas_call(
        _tables_body,
        out_shape=(jax.ShapeDtypeStruct((N_REL, D), jnp.float32),
                   jax.ShapeDtypeStruct((N_REL, D), jnp.float32)),
    )(edge_embs, Wr1, br1.reshape(1, D))


LBLK = 512


def _layer_body(apply_tanh, sd_ref, ss_ref, cd_ref, cs_ref, x_ref,
                wo_ref, bo_ref, wi_ref, bi_ref, ws_ref, bs_ref,
                bnw_ref, bnb_ref, o_ref):
    deg_d = jnp.maximum(cd_ref[:, 0:1], 1.0)
    deg_s = jnp.maximum(cs_ref[:, 0:1], 1.0)
    ho = sd_ref[...] / deg_d
    hi = ss_ref[...] / deg_s
    h = (jnp.dot(ho, wo_ref[...], preferred_element_type=jnp.float32) + bo_ref[...]
         + jnp.dot(hi, wi_ref[...], preferred_element_type=jnp.float32) + bi_ref[...]
         + jnp.dot(x_ref[...], ws_ref[...], preferred_element_type=jnp.float32)
         + bs_ref[...]) * (1.0 / 3.0)
    h = h * (bnw_ref[...] * _BN_SCALE) + bnb_ref[...]
    o_ref[...] = jnp.tanh(h) if apply_tanh else h


def _layer_tc(Sd, Ss, Cd, Cs, X, WO, bO, WI, bI, WS, bS, bnw, bnb, apply_tanh):
    grid = (NACC // LBLK,)
    row_spec = pl.BlockSpec((LBLK, D), lambda i: (i, 0))
    cnt_spec = pl.BlockSpec((LBLK, CNTW), lambda i: (i, 0))
    w_spec = pl.BlockSpec((D, D), lambda i: (0, 0))
    b_spec = pl.BlockSpec((1, D), lambda i: (0, 0))
    return pl.pallas_call(
        functools.partial(_layer_body, apply_tanh),
        grid=grid,
        in_specs=[row_spec, row_spec, cnt_spec, cnt_spec, row_spec,
                  w_spec, b_spec, w_spec, b_spec, w_spec, b_spec,
                  b_spec, b_spec],
        out_specs=row_spec,
        out_shape=jax.ShapeDtypeStruct((NACC, D), jnp.float32),
    )(Sd, Ss, Cd, Cs, X,
      WO, bO.reshape(1, D), WI, bI.reshape(1, D), WS, bS.reshape(1, D),
      bnw.reshape(1, D), bnb.reshape(1, D))


NPAD = 640
SBLK = 128
SGRID = NPAD // SBLK


def _score_body(hn_blk_ref, hn_full_ref, ee_ref, wr_ref, br_ref,
                hid_ref, rid_ref, tid_ref, ihp_ref, o_ref, c_ref):
    q = pl.program_id(0)
    j = pl.program_id(1)

    @pl.when(jnp.logical_and(q == 0, j == 0))
    def _():
        heads = jnp.concatenate(
            [hn_full_ref[pl.ds(hid_ref[0, b], 1), :] for b in range(B)], axis=0)
        tails = jnp.concatenate(
            [hn_full_ref[pl.ds(tid_ref[0, b], 1), :] for b in range(B)], axis=0)
        rrows = jnp.concatenate(
            [ee_ref[pl.ds(rid_ref[0, b], 1), :] for b in range(B)], axis=0)
        rels = jnp.dot(rrows, wr_ref[...],
                       preferred_element_type=jnp.float32) + br_ref[...]
        c_ref[...] = jnp.where(ihp_ref[0, 0] == 1, rels - tails,
                               -(heads + rels))

    cq = c_ref[pl.ds(q, 1), :]
    dist = jnp.sum(jnp.abs(hn_blk_ref[...] + cq), axis=1)
    o_ref[0, 0, :] = jax.nn.sigmoid(GAMMA - dist)


def _score_tc(hn_pad, edge_embs, Wr1, br1, hids, rids, tids, ihp):
    grid = (B, SGRID)
    smem = pl.BlockSpec(memory_space=pltpu.SMEM)
    return pl.pallas_call(
        _score_body,
        grid=grid,
        in_specs=[
            pl.BlockSpec((SBLK, D), lambda q, j: (q * SGRID + j, 0)),
            pl.BlockSpec((B * NPAD, D), lambda q, j: (0, 0)),
            pl.BlockSpec((N_REL, D), lambda q, j: (0, 0)),
            pl.BlockSpec((D, D), lambda q, j: (0, 0)),
            pl.BlockSpec((1, D), lambda q, j: (0, 0)),
            smem, smem, smem, smem,
        ],
        out_specs=pl.BlockSpec((1, 1, SBLK), lambda q, j: (q, 0, j)),
        out_shape=jax.ShapeDtypeStruct((B, 1, NPAD), jnp.float32),
        scratch_shapes=[pltpu.VMEM((B, D), jnp.float32)],
    )(hn_pad, hn_pad, edge_embs, Wr1, br1.reshape(1, D),
      hids.reshape(1, B), rids.reshape(1, B), tids.reshape(1, B),
      jnp.asarray(ihp, jnp.int32).reshape(1, 1))


def kernel(node_embs, edge_embs, edge_index, e_vid, hids, rids, tids,
           is_head_pred, WO1, bO1, WI1, bI1, WS1, bS1, bn1w, bn1b, Wr1, br1,
           WO2, bO2, WI2, bI2, WS2, bS2, bn2w, bn2b):
    src1d = edge_index[0].reshape(N_EDGES)
    dst1d = edge_index[1].reshape(N_EDGES)
    vid1d = e_vid.reshape(N_EDGES)
    gidx2 = jnp.concatenate([src1d, dst1d])
    sidx2 = jnp.concatenate([dst1d, src1d])

    negT1, negT2 = _rel_tables(edge_embs, Wr1, br1)

    x0 = jnp.pad(node_embs, ((0, NACC - N_NODES), (0, 0)))
    zrows = jnp.zeros((NSLICE, D), jnp.float32)
    ones_rows = jnp.ones((CH, D), jnp.float32)

    Cb = _sc_counts(sidx2, ones_rows, zrows)[0]
    Cd = Cb[0, :, :CNTW]
    Cs = Cb[1, :, :CNTW]

    S1 = _sc_layer(x0, negT1, gidx2, sidx2, vid1d, zrows)[0]
    h1 = _layer_tc(S1[0], S1[1], Cd, Cs, x0,
                   WO1, bO1, WI1, bI1, WS1, bS1, bn1w, bn1b, apply_tanh=True)

    S2 = _sc_layer(h1, negT2, gidx2, sidx2, vid1d, zrows)[0]
    hn = _layer_tc(S2[0], S2[1], Cd, Cs, h1,
                   WO2, bO2, WI2, bI2, WS2, bS2, bn2w, bn2b,
                   apply_tanh=False)[:N_NODES]

    hn_pad = jnp.pad(hn.reshape(B, N_NODES // B, D),
                     ((0, 0), (0, NPAD - N_NODES // B), (0, 0))
                     ).reshape(B * NPAD, D)
    P = _score_tc(hn_pad, edge_embs, Wr1, br1, hids, rids, tids,
                  is_head_pred).reshape(B, NPAD)
    score = jnp.repeat(P[:, :N_NODES // B], B, axis=1)
    return hn, score
